# baseline (speedup 1.0000x reference)
import functools

import jax
import jax.numpy as jnp
import numpy as np
from jax.experimental import pallas as pl
from jax.experimental.shard_map import shard_map
from jax.sharding import Mesh, PartitionSpec as P

_N, _IN_DIM, _HID, _CODE_DIM, _K = 2048, 512, 2048, 64, 1024
_BLK = 256


def _fused_body(x_ref, w1_ref, b1_ref, w2_ref, b2_ref, w3_ref, b3_ref,
                cb_ref, cbt_ref, out_ref, loss_ref):
    x = x_ref[...]
    h = jnp.maximum(
        jnp.dot(x, w1_ref[...], preferred_element_type=jnp.float32) + b1_ref[...], 0.0)
    h2 = jnp.dot(h, w2_ref[...], preferred_element_type=jnp.float32) + b2_ref[...]
    z = jnp.dot(h2, w3_ref[...], preferred_element_type=jnp.float32) + b3_ref[...]

    cb = cb_ref[...]
    zn = jnp.sum(z * z, axis=-1, keepdims=True)
    cn = jnp.sum(cb * cb, axis=-1)[None, :]
    zc = jnp.dot(z, cbt_ref[...], preferred_element_type=jnp.float32)
    d = (zn - 2.0 * zc) + cn

    m = jnp.min(d, axis=-1, keepdims=True)
    idx = jax.lax.broadcasted_iota(jnp.int32, d.shape, 1)
    words = jnp.min(jnp.where(d == m, idx, _K), axis=-1)
    onehot = (idx == words[:, None]).astype(jnp.float32)
    out_ref[...] = jnp.dot(onehot, cb, preferred_element_type=jnp.float32)

    @pl.when(pl.program_id(0) == 0)
    def _():
        loss_ref[...] = jnp.zeros((1, 1), jnp.float32)
    loss_ref[...] += 2.0 * jnp.sum(m).reshape(1, 1)


def _run_shard(x, W1, b1r, W2, b2r, W3, b3r, codebook, cbt):
    rows = x.shape[0]
    nblk = rows // _BLK
    out, loss = pl.pallas_call(
        _fused_body,
        grid=(nblk,),
        in_specs=[
            pl.BlockSpec((_BLK, _IN_DIM), lambda i: (i, 0)),
            pl.BlockSpec((_IN_DIM, _HID), lambda i: (0, 0)),
            pl.BlockSpec((1, _HID), lambda i: (0, 0)),
            pl.BlockSpec((_HID, _HID), lambda i: (0, 0)),
            pl.BlockSpec((1, _HID), lambda i: (0, 0)),
            pl.BlockSpec((_HID, _CODE_DIM), lambda i: (0, 0)),
            pl.BlockSpec((1, _CODE_DIM), lambda i: (0, 0)),
            pl.BlockSpec((_K, _CODE_DIM), lambda i: (0, 0)),
            pl.BlockSpec((_CODE_DIM, _K), lambda i: (0, 0)),
        ],
        out_specs=[
            pl.BlockSpec((_BLK, _CODE_DIM), lambda i: (i, 0)),
            pl.BlockSpec((1, 1), lambda i: (0, 0)),
        ],
        out_shape=[
            jax.ShapeDtypeStruct((rows, _CODE_DIM), jnp.float32),
            jax.ShapeDtypeStruct((1, 1), jnp.float32),
        ],
    )(x, W1, b1r, W2, b2r, W3, b3r, codebook, cbt)
    return out, loss


def kernel(x, W1, b1, W2, b2, W3, b3, codebook):
    cbt = codebook.T
    b1r, b2r, b3r = b1[None, :], b2[None, :], b3[None, :]
    devs = jax.devices()
    nd = max(d for d in (1, 2, 4, 8) if d <= len(devs))
    if nd == 1:
        out, loss = _run_shard(x, W1, b1r, W2, b2r, W3, b3r, codebook, cbt)
        return out, loss[0, 0]

    mesh = Mesh(np.array(devs[:nd]), ("d",))
    rep = P(None, None)

    def _sharded(x, W1, b1r, W2, b2r, W3, b3r, codebook, cbt):
        out, loss = _run_shard(x, W1, b1r, W2, b2r, W3, b3r, codebook, cbt)
        return out, jax.lax.psum(loss, "d")

    out, loss = shard_map(
        _sharded, mesh=mesh,
        in_specs=(P("d", None),) + (rep,) * 8,
        out_specs=(P("d", None), rep),
        check_rep=False,
    )(x, W1, b1r, W2, b2r, W3, b3r, codebook, cbt)
    return out, loss[0, 0]


# speedup vs baseline: 11.5526x; 11.5526x over previous
import jax
import jax.numpy as jnp
from jax.experimental import pallas as pl

_N, _IN_DIM, _HID, _CODE_DIM, _K = 2048, 512, 2048, 64, 1024
_BLK = 512


def _fused_body(x_ref, w1_ref, b1_ref, w2_ref, b2_ref, w3_ref, b3_ref,
                cb_ref, cbt_ref, out_ref, loss_ref):
    x = x_ref[...]
    h = jnp.maximum(
        jnp.dot(x, w1_ref[...], preferred_element_type=jnp.float32) + b1_ref[...], 0.0)
    h2 = jnp.dot(h, w2_ref[...], preferred_element_type=jnp.float32) + b2_ref[...]
    z = jnp.dot(h2, w3_ref[...], preferred_element_type=jnp.float32) + b3_ref[...]

    cb = cb_ref[...]
    zn = jnp.sum(z * z, axis=-1, keepdims=True)
    cn = jnp.sum(cb * cb, axis=-1)[None, :]
    zc = jnp.dot(z, cbt_ref[...], preferred_element_type=jnp.float32)
    d = (zn - 2.0 * zc) + cn

    m = jnp.min(d, axis=-1, keepdims=True)
    idx = jax.lax.broadcasted_iota(jnp.int32, d.shape, 1)
    words = jnp.min(jnp.where(d == m, idx, _K), axis=-1)
    onehot = (idx == words[:, None]).astype(jnp.float32)
    out_ref[...] = jnp.dot(onehot, cb, preferred_element_type=jnp.float32)

    @pl.when(pl.program_id(0) == 0)
    def _():
        loss_ref[...] = jnp.zeros((1, 1), jnp.float32)
    loss_ref[...] += 2.0 * jnp.sum(m).reshape(1, 1)


def kernel(x, W1, b1, W2, b2, W3, b3, codebook):
    nblk = _N // _BLK
    cbt = codebook.T
    b1r, b2r, b3r = b1[None, :], b2[None, :], b3[None, :]
    out, loss = pl.pallas_call(
        _fused_body,
        grid=(nblk,),
        in_specs=[
            pl.BlockSpec((_BLK, _IN_DIM), lambda i: (i, 0)),
            pl.BlockSpec((_IN_DIM, _HID), lambda i: (0, 0)),
            pl.BlockSpec((1, _HID), lambda i: (0, 0)),
            pl.BlockSpec((_HID, _HID), lambda i: (0, 0)),
            pl.BlockSpec((1, _HID), lambda i: (0, 0)),
            pl.BlockSpec((_HID, _CODE_DIM), lambda i: (0, 0)),
            pl.BlockSpec((1, _CODE_DIM), lambda i: (0, 0)),
            pl.BlockSpec((_K, _CODE_DIM), lambda i: (0, 0)),
            pl.BlockSpec((_CODE_DIM, _K), lambda i: (0, 0)),
        ],
        out_specs=[
            pl.BlockSpec((_BLK, _CODE_DIM), lambda i: (i, 0)),
            pl.BlockSpec((1, 1), lambda i: (0, 0)),
        ],
        out_shape=[
            jax.ShapeDtypeStruct((_N, _CODE_DIM), jnp.float32),
            jax.ShapeDtypeStruct((1, 1), jnp.float32),
        ],
    )(x, W1, b1r, W2, b2r, W3, b3r, codebook, cbt)
    return out, loss[0, 0]


# speedup vs baseline: 11.7993x; 1.0214x over previous
import jax
import jax.numpy as jnp
from jax.experimental import pallas as pl

_N, _IN_DIM, _HID, _CODE_DIM, _K = 2048, 512, 2048, 64, 1024
_BLK = 1024


def _fused_body(x_ref, w1_ref, b1_ref, w2_ref, b2_ref, w3_ref, b3_ref,
                cb_ref, cbt_ref, out_ref, loss_ref):
    x = x_ref[...]
    h = jnp.maximum(
        jnp.dot(x, w1_ref[...], preferred_element_type=jnp.float32) + b1_ref[...], 0.0)
    h2 = jnp.dot(h, w2_ref[...], preferred_element_type=jnp.float32) + b2_ref[...]
    z = jnp.dot(h2, w3_ref[...], preferred_element_type=jnp.float32) + b3_ref[...]

    cb = cb_ref[...]
    zn = jnp.sum(z * z, axis=-1, keepdims=True)
    cn = jnp.sum(cb * cb, axis=-1)[None, :]
    zc = jnp.dot(z, cbt_ref[...], preferred_element_type=jnp.float32)
    d = (zn - 2.0 * zc) + cn

    m = jnp.min(d, axis=-1, keepdims=True)
    idx = jax.lax.broadcasted_iota(jnp.int32, d.shape, 1)
    words = jnp.min(jnp.where(d == m, idx, _K), axis=-1)
    onehot = (idx == words[:, None]).astype(jnp.float32)
    out_ref[...] = jnp.dot(onehot, cb, preferred_element_type=jnp.float32)

    @pl.when(pl.program_id(0) == 0)
    def _():
        loss_ref[...] = jnp.zeros((1, 1), jnp.float32)
    loss_ref[...] += 2.0 * jnp.sum(m).reshape(1, 1)


def kernel(x, W1, b1, W2, b2, W3, b3, codebook):
    nblk = _N // _BLK
    cbt = codebook.T
    b1r, b2r, b3r = b1[None, :], b2[None, :], b3[None, :]
    out, loss = pl.pallas_call(
        _fused_body,
        grid=(nblk,),
        in_specs=[
            pl.BlockSpec((_BLK, _IN_DIM), lambda i: (i, 0)),
            pl.BlockSpec((_IN_DIM, _HID), lambda i: (0, 0)),
            pl.BlockSpec((1, _HID), lambda i: (0, 0)),
            pl.BlockSpec((_HID, _HID), lambda i: (0, 0)),
            pl.BlockSpec((1, _HID), lambda i: (0, 0)),
            pl.BlockSpec((_HID, _CODE_DIM), lambda i: (0, 0)),
            pl.BlockSpec((1, _CODE_DIM), lambda i: (0, 0)),
            pl.BlockSpec((_K, _CODE_DIM), lambda i: (0, 0)),
            pl.BlockSpec((_CODE_DIM, _K), lambda i: (0, 0)),
        ],
        out_specs=[
            pl.BlockSpec((_BLK, _CODE_DIM), lambda i: (i, 0)),
            pl.BlockSpec((1, 1), lambda i: (0, 0)),
        ],
        out_shape=[
            jax.ShapeDtypeStruct((_N, _CODE_DIM), jnp.float32),
            jax.ShapeDtypeStruct((1, 1), jnp.float32),
        ],
    )(x, W1, b1r, W2, b2r, W3, b3r, codebook, cbt)
    return out, loss[0, 0]
